# baseline (speedup 1.0000x reference)
import functools

import jax
import jax.numpy as jnp
from jax import lax
from jax.experimental import pallas as pl
from jax.experimental.pallas import tpu as pltpu
from jax.experimental.pallas import tpu_sc as plsc

N = 10000
E = 320000
IN_C = 128
EDGE_C = 4
HID = 64
OUT_C = 32
LAYER_NUM = 10

NW = 32
DPT = 320
NPAD = NW * DPT
EB = 128
UNROLL = 4
E_PAD = E + 4 * EB
NEG = float("-inf")


def _tc_pre_body(x_ref, we1_ref, we2_ref, wms_ref, wmd_ref, wux_ref, wpx_ref,
                 h_ref, a_ref, b_ref, ux_ref, px_ref):
    x = x_ref[...]
    h1 = jnp.maximum(jnp.dot(x, we1_ref[...], preferred_element_type=jnp.float32), 0.0)
    h = jnp.maximum(jnp.dot(h1, we2_ref[...], preferred_element_type=jnp.float32), 0.0)
    h_ref[...] = h
    a_ref[...] = jnp.dot(h, wms_ref[...], preferred_element_type=jnp.float32)
    b_ref[...] = jnp.dot(h, wmd_ref[...], preferred_element_type=jnp.float32)
    ux_ref[...] = jnp.dot(x, wux_ref[...], preferred_element_type=jnp.float32)
    px_ref[...] = jnp.dot(x, wpx_ref[...], preferred_element_type=jnp.float32)


def _tc_pre(x, we1, we2, wms, wmd, wux, wpx):
    out = [jax.ShapeDtypeStruct((N, HID), jnp.float32) for _ in range(5)]
    return pl.pallas_call(_tc_pre_body, out_shape=out)(x, we1, we2, wms, wmd, wux, wpx)


def _tc_layer_body(tmp_ref, b_ref, h_ref, ux_ref, px_ref,
                   wuh_ref, wua_ref, wroh_ref, wconf_ref, wms_ref, wmd_ref,
                   ho_ref, a_ref, b2_ref):
    tmp = tmp_ref[0:N, :]
    h = h_ref[...]
    agg = jnp.maximum(b_ref[...] + tmp, 0.0)
    hn = jnp.dot(h, wuh_ref[...], preferred_element_type=jnp.float32)
    hn = hn + jnp.dot(agg, wua_ref[...], preferred_element_type=jnp.float32)
    hn = jnp.maximum(hn + ux_ref[...], 0.0)
    r = jnp.dot(hn, wroh_ref[...], preferred_element_type=jnp.float32) + px_ref[...]
    g = jnp.maximum(jnp.max(r, axis=0, keepdims=True), 0.0)
    s = jnp.sum(g * wconf_ref[...])
    c = 1.0 / (1.0 + jnp.exp(-s))
    ho = c * hn + (1.0 - c) * h
    ho_ref[...] = ho
    a_ref[...] = jnp.dot(ho, wms_ref[...], preferred_element_type=jnp.float32)
    b2_ref[...] = jnp.dot(ho, wmd_ref[...], preferred_element_type=jnp.float32)


def _tc_layer(tmp, b, h, ux, px, wuh, wua, wroh, wconf, wms, wmd):
    out = [jax.ShapeDtypeStruct((N, HID), jnp.float32) for _ in range(3)]
    return pl.pallas_call(_tc_layer_body, out_shape=out)(
        tmp, b, h, ux, px, wuh, wua, wroh, wconf, wms, wmd)


def _tc_final_body(h_ref, px_ref, wroh_ref, whead_ref, wfin_ref, bfin_ref, out_ref):
    r = jnp.dot(h_ref[...], wroh_ref[...], preferred_element_type=jnp.float32) + px_ref[...]
    g = jnp.maximum(jnp.max(r, axis=0, keepdims=True), 0.0)
    o = jnp.dot(g, whead_ref[...], preferred_element_type=jnp.float32)
    z = jnp.dot(o, wfin_ref[...], preferred_element_type=jnp.float32) + bfin_ref[...]
    m = jnp.max(z, axis=-1, keepdims=True)
    lse = m + jnp.log(jnp.sum(jnp.exp(z - m), axis=-1, keepdims=True))
    out_ref[...] = z - lse


def _tc_final(h, px, wroh, whead, wfin, bfin):
    return pl.pallas_call(
        _tc_final_body,
        out_shape=jax.ShapeDtypeStruct((1, OUT_C), jnp.float32),
    )(h, px, wroh, whead, wfin, bfin)


EC_BLK = 1024


def _tc_ec_body(attrT_ref, wme_ref, ec_ref):
    ec_ref[...] = lax.dot_general(
        attrT_ref[...], wme_ref[...], (((0,), (0,)), ((), ())),
        preferred_element_type=jnp.float32)


def _tc_ec(attrT, wme):
    return pl.pallas_call(
        _tc_ec_body,
        grid=(E_PAD // EC_BLK,),
        in_specs=[pl.BlockSpec((EDGE_C, EC_BLK), lambda i: (0, i)),
                  pl.BlockSpec((EDGE_C, HID), lambda i: (0, 0))],
        out_specs=pl.BlockSpec((EC_BLK, HID), lambda i: (i, 0)),
        out_shape=jax.ShapeDtypeStruct((E_PAD, HID), jnp.float32),
    )(attrT, wme)


def _sc_edge_body(a_hbm, esrc_hbm, edst_hbm, eattrT_hbm, eoff_hbm, wme_hbm,
                  out_hbm,
                  shared, idxb, gidx, dstb, attrb, rowsb, outl, eoffv, wmev,
                  s_idx, s_dst, s_attr, s_gat):
    nc = 2
    sid = lax.axis_index("s")
    wid = sid * nc + lax.axis_index("c")
    d0 = wid * DPT

    @pl.when(sid == 0)
    def _():
        pltpu.sync_copy(a_hbm, shared)
    pltpu.sync_copy(eoff_hbm, eoffv)
    pltpu.sync_copy(wme_hbm, wmev)

    def _init(r, _):
        for j in range(2 * HID // 16):
            outl[r, pl.ds(16 * j, 16)] = jnp.full((16,), NEG, jnp.float32)
        return 0
    lax.fori_loop(0, DPT // 2 + 1, _init, 0)

    ev = eoffv[pl.ds(wid, 16)]
    e_lo = ev[0]
    e_hi = ev[1]
    e_al = pl.multiple_of(e_lo - lax.rem(e_lo, EB), EB)
    nblk = lax.div(e_hi - e_al + (EB - 1), EB)
    nblk = jnp.maximum(nblk, 1)

    wv = [[wmev[k, pl.ds(16 * j, 16)] for j in range(HID // 16)]
          for k in range(EDGE_C)]

    plsc.subcore_barrier()

    def _lin_start(blk, slot):
        start = pl.multiple_of(e_al + blk * EB, EB)
        pltpu.async_copy(esrc_hbm.at[pl.ds(start, EB)], idxb.at[slot], s_idx.at[slot])
        pltpu.async_copy(edst_hbm.at[pl.ds(start, EB)], dstb.at[slot], s_dst.at[slot])
        pltpu.async_copy(eattrT_hbm.at[:, pl.ds(start, EB)], attrb.at[slot], s_attr.at[slot])

    def _wait_idx(blk, slot):
        start = pl.multiple_of(e_al + blk * EB, EB)
        pltpu.make_async_copy(esrc_hbm.at[pl.ds(start, EB)], idxb.at[slot], s_idx.at[slot]).wait()

    def _wait_lin(blk, slot):
        start = pl.multiple_of(e_al + blk * EB, EB)
        pltpu.make_async_copy(edst_hbm.at[pl.ds(start, EB)], dstb.at[slot], s_dst.at[slot]).wait()
        pltpu.make_async_copy(eattrT_hbm.at[:, pl.ds(start, EB)], attrb.at[slot], s_attr.at[slot]).wait()

    def _gat_start(slot):
        for q in range(EB // 16):
            sl = pl.ds(16 * q, 16)
            gidx[slot, sl] = lax.shift_right_logical(idxb[slot, sl], 1)
        pltpu.async_copy(shared.at[gidx.at[slot]], rowsb.at[slot], s_gat.at[slot])

    def _gat_wait(slot):
        pltpu.make_async_copy(shared.at[gidx.at[slot]], rowsb.at[slot], s_gat.at[slot]).wait()

    _lin_start(0, 0)
    _wait_idx(0, 0)
    _gat_start(0)

    @pl.when(nblk > 1)
    def _():
        _lin_start(1, 1)

    def _flush(prev_lr, acc):
        pr = lax.shift_right_logical(prev_lr, 1)
        co = (prev_lr & 1) * HID
        for j in range(HID // 16):
            sl = pl.ds(co + 16 * j, 16)
            outl[pr, sl] = jnp.maximum(outl[pr, sl], acc[j])

    def _blk_body(blk, carry):
        prev_lr, acc = carry
        slot = lax.rem(blk, 2)
        other = 1 - slot

        _wait_lin(blk, slot)
        _gat_wait(slot)

        @pl.when(blk + 1 < nblk)
        def _():
            _wait_idx(blk + 1, other)
            _gat_start(other)

        def _edge_body(i, carry2):
            prev_lr, acc = carry2
            base = i * 16
            dvec = dstb[slot, pl.ds(base, 16)]
            svec = idxb[slot, pl.ds(base, 16)]
            av = [attrb[slot, k, pl.ds(base, 16)] for k in range(EDGE_C)]
            for u in range(16):
                e = base + u
                d = dvec[u]
                lr = d - d0
                bad = jnp.logical_or(lr < 0, lr >= DPT)
                lr = jnp.where(bad, DPT, lr)
                changed = lr != prev_lr

                @pl.when(changed)
                def _(prev_lr=prev_lr, acc=acc):
                    _flush(prev_lr, acc)

                po = (svec[u] & 1) * HID
                s0 = av[0][u]
                s1 = av[1][u]
                s2 = av[2][u]
                s3 = av[3][u]
                new_acc = []
                for j in range(HID // 16):
                    row = rowsb[slot, e, pl.ds(po + 16 * j, 16)]
                    t = wv[0][j] * s0 + wv[1][j] * s1 + wv[2][j] * s2 + wv[3][j] * s3
                    val = row + t
                    new_acc.append(jnp.where(changed, val, jnp.maximum(acc[j], val)))
                acc = tuple(new_acc)
                prev_lr = lr
            return prev_lr, acc

        carry = lax.fori_loop(0, EB // 16, _edge_body, (prev_lr, acc))

        @pl.when(blk + 2 < nblk)
        def _():
            _lin_start(blk + 2, slot)

        return carry

    acc0 = tuple(jnp.full((16,), NEG, jnp.float32) for _ in range(HID // 16))
    prev_lr, acc = lax.fori_loop(0, nblk, _blk_body, (jnp.int32(DPT), acc0))
    _flush(prev_lr, acc)

    pltpu.sync_copy(outl.at[pl.ds(0, DPT // 2)],
                    out_hbm.at[pl.ds(wid * (DPT // 2), DPT // 2)])


@functools.partial(jax.jit, static_argnames=("interpret",))
def _sc_edge(a2, esrc, edst, eattrT, eoff, wme, interpret=False):
    mesh = plsc.VectorSubcoreMesh(
        core_axis_name="c", subcore_axis_name="s", num_cores=2, num_subcores=16)
    f = pl.kernel(
        _sc_edge_body,
        out_type=jax.ShapeDtypeStruct((NPAD // 2, 2 * HID), jnp.float32),
        mesh=mesh,
        scratch_types=[
            pltpu.VMEM_SHARED((N // 2, 2 * HID), jnp.float32),
            pltpu.VMEM((2, EB), jnp.int32),
            pltpu.VMEM((2, EB), jnp.int32),
            pltpu.VMEM((2, EB), jnp.int32),
            pltpu.VMEM((2, EDGE_C, EB), jnp.float32),
            pltpu.VMEM((2, EB, 2 * HID), jnp.float32),
            pltpu.VMEM((DPT // 2 + 4, 2 * HID), jnp.float32),
            pltpu.VMEM((48,), jnp.int32),
            pltpu.VMEM((EDGE_C, HID), jnp.float32),
            pltpu.SemaphoreType.DMA((2,)),
            pltpu.SemaphoreType.DMA((2,)),
            pltpu.SemaphoreType.DMA((2,)),
            pltpu.SemaphoreType.DMA((2,)),
        ],
        interpret=interpret,
    )
    return f(a2, esrc, edst, eattrT, eoff, wme)


def kernel(x, edge_index, edge_attr, batch, W_emb1, W_emb2, W_msg, W_upd,
           W_ro, W_conf, W_head, W_fin, b_fin):
    del batch

    wms = W_msg[0:HID]
    wmd = W_msg[HID:2 * HID]
    wme = W_msg[2 * HID:]
    wuh = W_upd[0:HID]
    wua = W_upd[HID:2 * HID]
    wux = W_upd[2 * HID:]
    wroh = W_ro[0:HID]
    wrox = W_ro[HID:]
    wconf_row = W_conf.reshape(1, HID)
    bfin_row = b_fin.reshape(1, OUT_C)

    src = edge_index[0]
    dst = edge_index[1]
    sdst, ssrc, a0, a1, a2, a3 = lax.sort(
        (dst, src, edge_attr[:, 0], edge_attr[:, 1], edge_attr[:, 2],
         edge_attr[:, 3]), num_keys=1)
    sattr = jnp.stack([a0, a1, a2, a3], axis=0)
    pad = E_PAD - E
    esrc = jnp.concatenate([ssrc, jnp.zeros((pad,), jnp.int32)])
    edst = jnp.concatenate([sdst, jnp.full((pad,), jnp.int32(2 * N))])
    eattr = jnp.concatenate([sattr, jnp.zeros((EDGE_C, pad), jnp.float32)], axis=1)
    bounds = (jnp.arange(33, dtype=jnp.int32) * DPT)
    eoff = jnp.searchsorted(sdst, bounds, side="left").astype(jnp.int32)
    eoff = jnp.concatenate([eoff, jnp.zeros((15,), jnp.int32)])

    h, a, b, ux, px = _tc_pre(x, W_emb1, W_emb2, wms, wmd, wux, wrox)

    def body(_, carry):
        h, a, b = carry
        a2 = a.reshape(N // 2, 2 * HID)
        tmp2 = _sc_edge(a2, esrc, edst, eattr, eoff, wme)
        tmp = tmp2.reshape(NPAD, HID)
        h, a, b = _tc_layer(tmp, b, h, ux, px, wuh, wua, wroh, wconf_row, wms, wmd)
        return h, a, b

    h, a, b = lax.fori_loop(0, LAYER_NUM, body, (h, a, b))
    return _tc_final(h, px, wroh, W_head, W_fin, bfin_row)


# speedup vs baseline: 1.2985x; 1.2985x over previous
import functools

import jax
import jax.numpy as jnp
from jax import lax
from jax.experimental import pallas as pl
from jax.experimental.pallas import tpu as pltpu
from jax.experimental.pallas import tpu_sc as plsc

N = 10000
E = 320000
IN_C = 128
EDGE_C = 4
HID = 64
OUT_C = 32
LAYER_NUM = 10

NW = 32
DPT = 320
NPAD = NW * DPT
EB = 128
UNROLL = 4
E_PAD = E + 2 * EB
NEG = float("-inf")


def _tc_pre_body(x_ref, we1_ref, we2_ref, wms_ref, wmd_ref, wux_ref, wpx_ref,
                 h_ref, a_ref, b_ref, ux_ref, px_ref):
    x = x_ref[...]
    h1 = jnp.maximum(jnp.dot(x, we1_ref[...], preferred_element_type=jnp.float32), 0.0)
    h = jnp.maximum(jnp.dot(h1, we2_ref[...], preferred_element_type=jnp.float32), 0.0)
    h_ref[...] = h
    ah = jnp.dot(h, wms_ref[...], preferred_element_type=jnp.float32)
    a_ref[...] = jnp.concatenate([ah, jnp.zeros_like(ah)], axis=1)
    b_ref[...] = jnp.dot(h, wmd_ref[...], preferred_element_type=jnp.float32)
    ux_ref[...] = jnp.dot(x, wux_ref[...], preferred_element_type=jnp.float32)
    px_ref[...] = jnp.dot(x, wpx_ref[...], preferred_element_type=jnp.float32)


def _tc_pre(x, we1, we2, wms, wmd, wux, wpx):
    out = [jax.ShapeDtypeStruct((N, HID), jnp.float32),
           jax.ShapeDtypeStruct((N, 2 * HID), jnp.float32),
           jax.ShapeDtypeStruct((N, HID), jnp.float32),
           jax.ShapeDtypeStruct((N, HID), jnp.float32),
           jax.ShapeDtypeStruct((N, HID), jnp.float32)]
    return pl.pallas_call(_tc_pre_body, out_shape=out)(x, we1, we2, wms, wmd, wux, wpx)


def _tc_layer_body(tmp_ref, b_ref, h_ref, ux_ref, px_ref,
                   wuh_ref, wua_ref, wroh_ref, wconf_ref, wms_ref, wmd_ref,
                   ho_ref, a_ref, b2_ref):
    tmp = tmp_ref[0:N, :]
    h = h_ref[...]
    agg = jnp.maximum(b_ref[...] + tmp, 0.0)
    hn = jnp.dot(h, wuh_ref[...], preferred_element_type=jnp.float32)
    hn = hn + jnp.dot(agg, wua_ref[...], preferred_element_type=jnp.float32)
    hn = jnp.maximum(hn + ux_ref[...], 0.0)
    r = jnp.dot(hn, wroh_ref[...], preferred_element_type=jnp.float32) + px_ref[...]
    g = jnp.maximum(jnp.max(r, axis=0, keepdims=True), 0.0)
    s = jnp.sum(g * wconf_ref[...])
    c = 1.0 / (1.0 + jnp.exp(-s))
    ho = c * hn + (1.0 - c) * h
    ho_ref[...] = ho
    an = jnp.dot(ho, wms_ref[...], preferred_element_type=jnp.float32)
    a_ref[...] = jnp.concatenate([an, jnp.zeros_like(an)], axis=1)
    b2_ref[...] = jnp.dot(ho, wmd_ref[...], preferred_element_type=jnp.float32)


def _tc_layer(tmp, b, h, ux, px, wuh, wua, wroh, wconf, wms, wmd):
    out = [jax.ShapeDtypeStruct((N, HID), jnp.float32),
           jax.ShapeDtypeStruct((N, 2 * HID), jnp.float32),
           jax.ShapeDtypeStruct((N, HID), jnp.float32)]
    return pl.pallas_call(_tc_layer_body, out_shape=out)(
        tmp, b, h, ux, px, wuh, wua, wroh, wconf, wms, wmd)


def _tc_final_body(h_ref, px_ref, wroh_ref, whead_ref, wfin_ref, bfin_ref, out_ref):
    r = jnp.dot(h_ref[...], wroh_ref[...], preferred_element_type=jnp.float32) + px_ref[...]
    g = jnp.maximum(jnp.max(r, axis=0, keepdims=True), 0.0)
    o = jnp.dot(g, whead_ref[...], preferred_element_type=jnp.float32)
    z = jnp.dot(o, wfin_ref[...], preferred_element_type=jnp.float32) + bfin_ref[...]
    m = jnp.max(z, axis=-1, keepdims=True)
    lse = m + jnp.log(jnp.sum(jnp.exp(z - m), axis=-1, keepdims=True))
    out_ref[...] = z - lse


def _tc_final(h, px, wroh, whead, wfin, bfin):
    return pl.pallas_call(
        _tc_final_body,
        out_shape=jax.ShapeDtypeStruct((1, OUT_C), jnp.float32),
    )(h, px, wroh, whead, wfin, bfin)


def _sc_edge_body(a_hbm, esrc_hbm, edst_hbm, eattr_hbm, eoff_hbm, wme_hbm,
                  out_hbm,
                  idxb, dstb, attrb, rowsb, outl, eoffv, wmev,
                  s_idx, s_dst, s_attr, s_gat):
    nc = 2
    wid = lax.axis_index("s") * nc + lax.axis_index("c")
    d0 = wid * DPT

    pltpu.sync_copy(eoff_hbm, eoffv)
    pltpu.sync_copy(wme_hbm, wmev)

    def _init(r, _):
        for j in range(HID // 16):
            outl[r, pl.ds(16 * j, 16)] = jnp.full((16,), NEG, jnp.float32)
        return 0
    lax.fori_loop(0, DPT + 1, _init, 0)

    ev = eoffv[pl.ds(wid, 16)]
    e_lo = ev[0]
    e_hi = ev[1]
    e_al = pl.multiple_of(e_lo - lax.rem(e_lo, EB), EB)
    nblk = lax.div(e_hi - e_al + (EB - 1), EB)
    nblk = jnp.maximum(nblk, 1)

    wv = [[wmev[k, pl.ds(16 * j, 16)] for j in range(HID // 16)]
          for k in range(EDGE_C)]

    def _lin_start(blk, slot):
        start = pl.multiple_of(e_al + blk * EB, EB)
        pltpu.async_copy(esrc_hbm.at[pl.ds(start, EB)], idxb.at[slot], s_idx.at[slot])
        pltpu.async_copy(edst_hbm.at[pl.ds(start, EB)], dstb.at[slot], s_dst.at[slot])
        pltpu.async_copy(eattr_hbm.at[:, pl.ds(start, EB)], attrb.at[slot], s_attr.at[slot])

    def _wait_idx(blk, slot):
        start = pl.multiple_of(e_al + blk * EB, EB)
        pltpu.make_async_copy(esrc_hbm.at[pl.ds(start, EB)], idxb.at[slot], s_idx.at[slot]).wait()

    def _wait_lin(blk, slot):
        start = pl.multiple_of(e_al + blk * EB, EB)
        pltpu.make_async_copy(edst_hbm.at[pl.ds(start, EB)], dstb.at[slot], s_dst.at[slot]).wait()
        pltpu.make_async_copy(eattr_hbm.at[:, pl.ds(start, EB)], attrb.at[slot], s_attr.at[slot]).wait()

    def _gat_start(slot):
        pltpu.async_copy(a_hbm.at[idxb.at[slot]], rowsb.at[slot], s_gat.at[slot])

    def _gat_wait(slot):
        pltpu.make_async_copy(a_hbm.at[idxb.at[slot]], rowsb.at[slot], s_gat.at[slot]).wait()

    _lin_start(0, 0)
    _wait_idx(0, 0)
    _gat_start(0)

    @pl.when(nblk > 1)
    def _():
        _lin_start(1, 1)

    def _flush(prev_lr, acc):
        for j in range(HID // 16):
            sl = pl.ds(16 * j, 16)
            outl[prev_lr, sl] = jnp.maximum(outl[prev_lr, sl], acc[j])

    def _blk_body(blk, carry):
        prev_lr, acc = carry
        slot = lax.rem(blk, 2)
        other = 1 - slot

        _wait_lin(blk, slot)
        _gat_wait(slot)

        @pl.when(blk + 1 < nblk)
        def _():
            _wait_idx(blk + 1, other)
            _gat_start(other)

        def _edge_body(i, carry2):
            prev_lr, acc = carry2
            base = i * 16
            dvec = dstb[slot, pl.ds(base, 16)]
            av = [attrb[slot, k, pl.ds(base, 16)] for k in range(EDGE_C)]
            for u in range(16):
                e = base + u
                d = dvec[u]
                lr = d - d0
                bad = jnp.logical_or(lr < 0, lr >= DPT)
                lr = jnp.where(bad, DPT, lr)
                changed = lr != prev_lr

                @pl.when(changed)
                def _(prev_lr=prev_lr, acc=acc):
                    _flush(prev_lr, acc)

                s0 = av[0][u]
                s1 = av[1][u]
                s2 = av[2][u]
                s3 = av[3][u]
                new_acc = []
                for j in range(HID // 16):
                    row = rowsb[slot, e, pl.ds(16 * j, 16)]
                    t = wv[0][j] * s0 + wv[1][j] * s1 + wv[2][j] * s2 + wv[3][j] * s3
                    val = row + t
                    new_acc.append(jnp.where(changed, val, jnp.maximum(acc[j], val)))
                acc = tuple(new_acc)
                prev_lr = lr
            return prev_lr, acc

        carry = lax.fori_loop(0, EB // 16, _edge_body, (prev_lr, acc))

        @pl.when(blk + 2 < nblk)
        def _():
            _lin_start(blk + 2, slot)

        return carry

    acc0 = tuple(jnp.full((16,), NEG, jnp.float32) for _ in range(HID // 16))
    prev_lr, acc = lax.fori_loop(0, nblk, _blk_body, (jnp.int32(DPT), acc0))
    _flush(prev_lr, acc)

    pltpu.sync_copy(outl.at[pl.ds(0, DPT)], out_hbm.at[pl.ds(d0, DPT)])


@functools.partial(jax.jit, static_argnames=("interpret",))
def _sc_edge(a, esrc, edst, eattr, eoff, wme, interpret=False):
    mesh = plsc.VectorSubcoreMesh(
        core_axis_name="c", subcore_axis_name="s", num_cores=2, num_subcores=16)
    f = pl.kernel(
        _sc_edge_body,
        out_type=jax.ShapeDtypeStruct((NPAD, HID), jnp.float32),
        mesh=mesh,
        scratch_types=[
            pltpu.VMEM((2, EB), jnp.int32),
            pltpu.VMEM((2, EB), jnp.int32),
            pltpu.VMEM((2, EDGE_C, EB), jnp.float32),
            pltpu.VMEM((2, EB, 2 * HID), jnp.float32),
            pltpu.VMEM((DPT + 8, HID), jnp.float32),
            pltpu.VMEM((48,), jnp.int32),
            pltpu.VMEM((EDGE_C, HID), jnp.float32),
            pltpu.SemaphoreType.DMA((2,)),
            pltpu.SemaphoreType.DMA((2,)),
            pltpu.SemaphoreType.DMA((2,)),
            pltpu.SemaphoreType.DMA((2,)),
        ],
        interpret=interpret,
    )
    return f(a, esrc, edst, eattr, eoff, wme)


def kernel(x, edge_index, edge_attr, batch, W_emb1, W_emb2, W_msg, W_upd,
           W_ro, W_conf, W_head, W_fin, b_fin):
    del batch

    wms = W_msg[0:HID]
    wmd = W_msg[HID:2 * HID]
    wme = W_msg[2 * HID:]
    wuh = W_upd[0:HID]
    wua = W_upd[HID:2 * HID]
    wux = W_upd[2 * HID:]
    wroh = W_ro[0:HID]
    wrox = W_ro[HID:]
    wconf_row = W_conf.reshape(1, HID)
    bfin_row = b_fin.reshape(1, OUT_C)

    src = edge_index[0]
    dst = edge_index[1]
    eid = jnp.arange(E, dtype=jnp.int32)
    sdst, sid = lax.sort((dst, eid), num_keys=1)
    ssrc = jnp.take(src, sid)
    sattr = jnp.take(edge_attr, sid, axis=0).T
    pad = E_PAD - E
    esrc = jnp.concatenate([ssrc, jnp.zeros((pad,), jnp.int32)])
    edst = jnp.concatenate([sdst, jnp.full((pad,), jnp.int32(2 * N))])
    eattr = jnp.concatenate([sattr, jnp.zeros((EDGE_C, pad), jnp.float32)], axis=1)
    bounds = (jnp.arange(33, dtype=jnp.int32) * DPT)
    eoff = jnp.searchsorted(sdst, bounds, side="left").astype(jnp.int32)
    eoff = jnp.concatenate([eoff, jnp.zeros((15,), jnp.int32)])

    h, a, b, ux, px = _tc_pre(x, W_emb1, W_emb2, wms, wmd, wux, wrox)

    def body(_, carry):
        h, a, b = carry
        tmp = _sc_edge(a, esrc, edst, eattr, eoff, wme)
        h, a, b = _tc_layer(tmp, b, h, ux, px, wuh, wua, wroh, wconf_row, wms, wmd)
        return h, a, b

    h, a, b = lax.fori_loop(0, LAYER_NUM, body, (h, a, b))
    return _tc_final(h, px, wroh, W_head, W_fin, bfin_row)
